# baseline (speedup 1.0000x reference)
import jax
import jax.numpy as jnp
from jax import lax
from jax.experimental import pallas as pl
from jax.experimental.pallas import tpu as pltpu

_TOPK = 5
_B = 4
_V = 32
_PD = 7
_N = _B * _TOPK
_NEG = -1e30
_BIG = 1 << 30


def _body(sel_ref, pose_ref, img_hbm, out_pose_hbm, out_prob_hbm,
          out_img_hbm, buf, in_sems, out_sems, pose_v, prob_v, small_sem):
    w = sel_ref[...]
    iotac = lax.broadcasted_iota(jnp.int32, (_B, _V), 1)

    idx_cols, val_cols = [], []
    in_copies = [None] * _N
    for t in range(_TOPK):
        m = jnp.max(w, axis=1, keepdims=True)
        eq = w == m
        idxc = jnp.min(jnp.where(eq, iotac, _BIG), axis=1, keepdims=True)
        idx_cols.append(idxc)
        val_cols.append(m)
        w = jnp.where(iotac == idxc, _NEG, w)
        for b in range(_B):
            j = b * _TOPK + t
            cp = pltpu.make_async_copy(
                img_hbm.at[b, idxc[b, 0]], buf.at[j], in_sems.at[j])
            cp.start()
            in_copies[j] = cp

    vals = jnp.concatenate(val_cols, axis=1)
    e = jnp.exp(vals - val_cols[0])
    prob_v[...] = e / jnp.sum(e, axis=1, keepdims=True)
    prob_cp = pltpu.make_async_copy(prob_v, out_prob_hbm, small_sem)
    prob_cp.start()

    poses_t = pose_ref[...]
    for t in range(_TOPK):
        oh = (iotac == idx_cols[t]).astype(jnp.float32)
        pose_v[:, :, t] = jnp.sum(oh[None, :, :] * poses_t, axis=2)
    pose_cp = pltpu.make_async_copy(pose_v, out_pose_hbm, small_sem)
    pose_cp.start()

    out_copies = []
    for b in range(_B):
        for t in range(_TOPK):
            in_copies[b * _TOPK + t].wait()
        oc = pltpu.make_async_copy(
            buf.at[pl.ds(b * _TOPK, _TOPK)], out_img_hbm.at[b],
            out_sems.at[b])
        oc.start()
        out_copies.append(oc)
    prob_cp.wait()
    pose_cp.wait()
    for oc in out_copies:
        oc.wait()


_call = pl.pallas_call(
    _body,
    grid_spec=pltpu.PrefetchScalarGridSpec(
        num_scalar_prefetch=0,
        grid=(),
        in_specs=[
            pl.BlockSpec(memory_space=pltpu.VMEM),
            pl.BlockSpec(memory_space=pltpu.VMEM),
            pl.BlockSpec(memory_space=pltpu.MemorySpace.HBM),
        ],
        out_specs=[
            pl.BlockSpec(memory_space=pltpu.MemorySpace.HBM),
            pl.BlockSpec(memory_space=pltpu.MemorySpace.HBM),
            pl.BlockSpec(memory_space=pltpu.MemorySpace.HBM),
        ],
        scratch_shapes=[
            pltpu.VMEM((_N, 3, 128, 128), jnp.float32),
            pltpu.SemaphoreType.DMA((_N,)),
            pltpu.SemaphoreType.DMA((_B,)),
            pltpu.VMEM((_PD, _B, _TOPK), jnp.float32),
            pltpu.VMEM((_B, _TOPK), jnp.float32),
            pltpu.SemaphoreType.DMA,
        ],
    ),
    out_shape=(
        jax.ShapeDtypeStruct((_PD, _B, _TOPK), jnp.float32),
        jax.ShapeDtypeStruct((_B, _TOPK), jnp.float32),
        jax.ShapeDtypeStruct((_B, _TOPK, 3, 128, 128), jnp.float32),
    ),
)


@jax.jit
def kernel(selection_weights, images, poses):
    imgs_t = jnp.transpose(images, (0, 1, 4, 2, 3))
    poses_t = jnp.transpose(poses, (2, 0, 1))
    out_pose_t, out_prob, out_img_t = _call(selection_weights, poses_t, imgs_t)
    return (
        jnp.transpose(out_img_t, (0, 1, 3, 4, 2)),
        jnp.transpose(out_pose_t, (1, 2, 0)),
        out_prob,
    )


# speedup vs baseline: 1.0021x; 1.0021x over previous
import jax
import jax.numpy as jnp
from jax import lax
from jax.experimental import pallas as pl
from jax.experimental.pallas import tpu as pltpu

_TOPK = 5
_B = 4
_V = 32
_PD = 7
_N = _B * _TOPK
_NEG = -1e30
_BIG = 1 << 30


def _body(sel_ref, pose_ref, img_hbm, out_pose_hbm, out_prob_hbm,
          out_img_hbm, buf, in_sems, out_sems, pose_v, prob_v, small_sem):
    w = sel_ref[...]
    iotac = lax.broadcasted_iota(jnp.int32, (_B, _V), 1)

    idx_cols, val_cols = [], []
    in_copies = [None] * _N
    for t in range(_TOPK):
        m = jnp.max(w, axis=1, keepdims=True)
        eq = w == m
        idxc = jnp.min(jnp.where(eq, iotac, _BIG), axis=1, keepdims=True)
        idx_cols.append(idxc)
        val_cols.append(m)
        w = jnp.where(iotac == idxc, _NEG, w)
        for b in range(_B):
            j = b * _TOPK + t
            cp = pltpu.make_async_copy(
                img_hbm.at[b, idxc[b, 0]], buf.at[j], in_sems.at[j])
            cp.start()
            in_copies[j] = cp

    vals = jnp.concatenate(val_cols, axis=1)
    e = jnp.exp(vals - val_cols[0])
    prob_v[...] = e / jnp.sum(e, axis=1, keepdims=True)
    prob_cp = pltpu.make_async_copy(prob_v, out_prob_hbm, small_sem)
    prob_cp.start()

    poses_t = pose_ref[...]
    for t in range(_TOPK):
        oh = (iotac == idx_cols[t]).astype(jnp.float32)
        pose_v[:, :, t] = jnp.sum(oh[None, :, :] * poses_t, axis=2)
    pose_cp = pltpu.make_async_copy(pose_v, out_pose_hbm, small_sem)
    pose_cp.start()

    out_copies = []
    for j, cp in enumerate(in_copies):
        cp.wait()
        b, t = divmod(j, _TOPK)
        oc = pltpu.make_async_copy(
            buf.at[j], out_img_hbm.at[b, t], out_sems.at[j])
        oc.start()
        out_copies.append(oc)
    prob_cp.wait()
    pose_cp.wait()
    for oc in out_copies:
        oc.wait()


_call = pl.pallas_call(
    _body,
    grid_spec=pltpu.PrefetchScalarGridSpec(
        num_scalar_prefetch=0,
        grid=(),
        in_specs=[
            pl.BlockSpec(memory_space=pltpu.VMEM),
            pl.BlockSpec(memory_space=pltpu.VMEM),
            pl.BlockSpec(memory_space=pltpu.MemorySpace.HBM),
        ],
        out_specs=[
            pl.BlockSpec(memory_space=pltpu.MemorySpace.HBM),
            pl.BlockSpec(memory_space=pltpu.MemorySpace.HBM),
            pl.BlockSpec(memory_space=pltpu.MemorySpace.HBM),
        ],
        scratch_shapes=[
            pltpu.VMEM((_N, 3, 128, 128), jnp.float32),
            pltpu.SemaphoreType.DMA((_N,)),
            pltpu.SemaphoreType.DMA((_N,)),
            pltpu.VMEM((_PD, _B, _TOPK), jnp.float32),
            pltpu.VMEM((_B, _TOPK), jnp.float32),
            pltpu.SemaphoreType.DMA,
        ],
    ),
    out_shape=(
        jax.ShapeDtypeStruct((_PD, _B, _TOPK), jnp.float32),
        jax.ShapeDtypeStruct((_B, _TOPK), jnp.float32),
        jax.ShapeDtypeStruct((_B, _TOPK, 3, 128, 128), jnp.float32),
    ),
)


@jax.jit
def kernel(selection_weights, images, poses):
    imgs_t = jnp.transpose(images, (0, 1, 4, 2, 3))
    poses_t = jnp.transpose(poses, (2, 0, 1))
    out_pose_t, out_prob, out_img_t = _call(selection_weights, poses_t, imgs_t)
    return (
        jnp.transpose(out_img_t, (0, 1, 3, 4, 2)),
        jnp.transpose(out_pose_t, (1, 2, 0)),
        out_prob,
    )


# speedup vs baseline: 1.0111x; 1.0089x over previous
import jax
import jax.numpy as jnp
from jax import lax
from jax.experimental import pallas as pl
from jax.experimental.pallas import tpu as pltpu

_TOPK = 5
_B = 4
_V = 32
_PD = 7
_N = _B * _TOPK
_NEG = -1e30
_BIG = 1 << 30


def _body(sel_ref, pose_hbm, img_hbm, out_pose_hbm, out_prob_hbm,
          out_img_hbm, buf, in_sems, out_sems, pose_v, prob_v, small_sem,
          pose_in_v, pose_in_sem):
    pose_in_cp = pltpu.make_async_copy(pose_hbm, pose_in_v, pose_in_sem)
    pose_in_cp.start()
    w = sel_ref[...]
    iotac = lax.broadcasted_iota(jnp.int32, (_B, _V), 1)

    idx_cols, val_cols = [], []
    in_copies = [None] * _N
    for t in range(_TOPK):
        m = jnp.max(w, axis=1, keepdims=True)
        eq = w == m
        idxc = jnp.min(jnp.where(eq, iotac, _BIG), axis=1, keepdims=True)
        idx_cols.append(idxc)
        val_cols.append(m)
        w = jnp.where(iotac == idxc, _NEG, w)
        for b in range(_B):
            j = b * _TOPK + t
            cp = pltpu.make_async_copy(
                img_hbm.at[b, idxc[b, 0]], buf.at[j], in_sems.at[j])
            cp.start()
            in_copies[j] = cp

    vals = jnp.concatenate(val_cols, axis=1)
    e = jnp.exp(vals - val_cols[0])
    prob_v[...] = e / jnp.sum(e, axis=1, keepdims=True)
    prob_cp = pltpu.make_async_copy(prob_v, out_prob_hbm, small_sem)
    prob_cp.start()

    pose_in_cp.wait()
    poses_t = pose_in_v[...]
    for t in range(_TOPK):
        oh = (iotac == idx_cols[t]).astype(jnp.float32)
        pose_v[:, :, t] = jnp.sum(oh[None, :, :] * poses_t, axis=2)
    pose_cp = pltpu.make_async_copy(pose_v, out_pose_hbm, small_sem)
    pose_cp.start()

    out_copies = []
    for j, cp in enumerate(in_copies):
        cp.wait()
        b, t = divmod(j, _TOPK)
        oc = pltpu.make_async_copy(
            buf.at[j], out_img_hbm.at[b, t], out_sems.at[j])
        oc.start()
        out_copies.append(oc)
    prob_cp.wait()
    pose_cp.wait()
    for oc in out_copies:
        oc.wait()


_call = pl.pallas_call(
    _body,
    grid_spec=pltpu.PrefetchScalarGridSpec(
        num_scalar_prefetch=0,
        grid=(),
        in_specs=[
            pl.BlockSpec(memory_space=pltpu.VMEM),
            pl.BlockSpec(memory_space=pltpu.MemorySpace.HBM),
            pl.BlockSpec(memory_space=pltpu.MemorySpace.HBM),
        ],
        out_specs=[
            pl.BlockSpec(memory_space=pltpu.MemorySpace.HBM),
            pl.BlockSpec(memory_space=pltpu.MemorySpace.HBM),
            pl.BlockSpec(memory_space=pltpu.MemorySpace.HBM),
        ],
        scratch_shapes=[
            pltpu.VMEM((_N, 3, 128, 128), jnp.float32),
            pltpu.SemaphoreType.DMA((_N,)),
            pltpu.SemaphoreType.DMA((_N,)),
            pltpu.VMEM((_PD, _B, _TOPK), jnp.float32),
            pltpu.VMEM((_B, _TOPK), jnp.float32),
            pltpu.SemaphoreType.DMA,
            pltpu.VMEM((_PD, _B, _V), jnp.float32),
            pltpu.SemaphoreType.DMA,
        ],
    ),
    out_shape=(
        jax.ShapeDtypeStruct((_PD, _B, _TOPK), jnp.float32),
        jax.ShapeDtypeStruct((_B, _TOPK), jnp.float32),
        jax.ShapeDtypeStruct((_B, _TOPK, 3, 128, 128), jnp.float32),
    ),
)


@jax.jit
def kernel(selection_weights, images, poses):
    imgs_t = jnp.transpose(images, (0, 1, 4, 2, 3))
    poses_t = jnp.transpose(poses, (2, 0, 1))
    out_pose_t, out_prob, out_img_t = _call(selection_weights, poses_t, imgs_t)
    return (
        jnp.transpose(out_img_t, (0, 1, 3, 4, 2)),
        jnp.transpose(out_pose_t, (1, 2, 0)),
        out_prob,
    )
